# baseline (speedup 1.0000x reference)
import dataclasses
import functools

import jax
import jax.numpy as jnp
from jax import lax
from jax.experimental import pallas as pl
from jax.experimental.pallas import tpu as pltpu
from jax.experimental.pallas import tpu_sc as plsc

K_SEL = 256
LN_EPS = 1e-5
_PREC = jax.lax.Precision.DEFAULT
_N_ITERS = 18


def _encode_body(x_ref, w_ref, lat_ref, mn_ref, mx_ref, mu_ref, std_ref,
                 xn_ref):
    step = pl.program_id(0)
    d = x_ref.shape[1]

    @pl.when(step == 0)
    def _():
        x = x_ref[...]
        mu = jnp.mean(x, axis=1, keepdims=True)
        xc = x - mu
        m2 = jnp.mean(xc, axis=1, keepdims=True)
        var = jnp.sum((xc - m2) * (xc - m2), axis=1, keepdims=True) / (d - 1)
        std = jnp.sqrt(var)
        xn_ref[...] = xc / (std + LN_EPS)
        mu_ref[...] = mu
        std_ref[...] = std

    w = w_ref[...]
    w2 = jnp.sum(w * w, axis=1, keepdims=True)
    inv = 1.0 / jnp.maximum(jnp.sqrt(w2), 1e-12)
    wn = w * inv
    lat = jax.lax.dot_general(
        xn_ref[...], wn, (((1,), (1,)), ((), ())),
        precision=_PREC, preferred_element_type=jnp.float32)
    lat_ref[...] = lat
    tmin = jnp.min(lat, axis=1, keepdims=True)
    tmax = jnp.max(lat, axis=1, keepdims=True)

    @pl.when(step == 0)
    def _():
        mn_ref[...] = tmin
        mx_ref[...] = tmax

    @pl.when(step > 0)
    def _():
        mn_ref[...] = jnp.minimum(mn_ref[...], tmin)
        mx_ref[...] = jnp.maximum(mx_ref[...], tmax)


def _select_body(lat_ref, mn_ref, mx_ref, thr_ref):
    b, h = lat_ref.shape
    n_chunks = 16
    ch = h // n_chunks

    def count_ge(t):
        c = jnp.zeros((b, 1), jnp.int32)
        for j in range(n_chunks):
            v = lat_ref[:, pl.ds(j * ch, ch)]
            c = c + jnp.sum((v >= t).astype(jnp.int32), axis=1,
                            keepdims=True)
        return c

    mn = mn_ref[...]
    mx = mx_ref[...]
    lo = mn
    hi = mx + (jnp.abs(mx) * 1e-6 + 1e-30)
    c_lo = jnp.full((b, 1), h, jnp.int32)
    c_hi = jnp.zeros((b, 1), jnp.int32)

    for _ in range(_N_ITERS):
        c_lo_f = c_lo.astype(jnp.float32)
        c_hi_f = jnp.maximum(c_hi.astype(jnp.float32), 0.7)
        frac = jnp.log(c_lo_f / K_SEL) / jnp.log(c_lo_f / c_hi_f)
        frac = jnp.clip(frac, 0.015625, 0.984375)
        m0 = lo + (hi - lo) * frac
        m = jnp.where((m0 > lo) & (m0 < hi), m0, 0.5 * (lo + hi))
        c = count_ge(m)
        go_hi = c <= K_SEL
        go_lo = c > K_SEL
        hi = jnp.where(go_hi, m, hi)
        c_hi = jnp.where(go_hi, c, c_hi)
        lo = jnp.where(go_lo, m, lo)
        c_lo = jnp.where(go_lo, c, c_lo)

    thr_ref[...] = hi


def _decode_body(lat_ref, w_ref, thr_ref, mu_ref, std_ref, out_ref, acc_ref):
    step = pl.program_id(0)
    nt = pl.num_programs(0)

    @pl.when(step == 0)
    def _():
        acc_ref[...] = jnp.zeros_like(acc_ref)

    lat = lat_ref[...]
    masked = jnp.where(lat >= thr_ref[...], lat, 0.0)
    acc_ref[...] += jax.lax.dot_general(
        masked, w_ref[...], (((1,), (0,)), ((), ())),
        precision=_PREC, preferred_element_type=jnp.float32)

    @pl.when(step == nt - 1)
    def _():
        out_ref[...] = acc_ref[...] * std_ref[...] + mu_ref[...]


_SC_ROWW = 16384
_SC_GW = 64


def _sc_decode(lat, thr, mu, std, W_enc):
    b, h = lat.shape
    d = W_enc.shape[1]
    f32 = jnp.float32
    i32 = jnp.int32
    mesh = plsc.VectorSubcoreMesh(core_axis_name="c", subcore_axis_name="s")
    n_tec = 32
    rows_per = b // n_tec
    nw = h // _SC_ROWW
    ng = K_SEL // _SC_GW
    nchunk = d // 16

    cp = pltpu.CompilerParams()
    if "needs_layout_passes" in pltpu.CompilerParams.__dataclass_fields__:
        cp = dataclasses.replace(cp, needs_layout_passes=False)

    @functools.partial(
        pl.kernel,
        out_type=jax.ShapeDtypeStruct((b, d), f32),
        mesh=mesh,
        compiler_params=cp,
        scratch_types=[
            pltpu.VMEM((_SC_ROWW,), f32),
            pltpu.VMEM((K_SEL,), f32),
            pltpu.VMEM((K_SEL,), i32),
            pltpu.VMEM((_SC_GW, d), f32),
            pltpu.VMEM((d,), f32),
            pltpu.VMEM((b,), f32),
            pltpu.VMEM((b,), f32),
            pltpu.VMEM((b,), f32),
            pltpu.VMEM((16,), i32),
            pltpu.SemaphoreType.DMA,
        ],
    )
    def sc_kernel(lat_hbm, thr_hbm, mu_hbm, std_hbm, w_hbm, out_hbm,
                  rowb, sel_val, sel_idx, wbuf, outv, thr_v, mu_v, std_v,
                  off_ref, sem):
        wid = lax.axis_index("s") * 2 + lax.axis_index("c")
        pltpu.sync_copy(thr_hbm, thr_v)
        pltpu.sync_copy(mu_hbm, mu_v)
        pltpu.sync_copy(std_hbm, std_v)
        ii = lax.iota(i32, 16)
        zf = jnp.zeros((16,), f32)
        zi = jnp.zeros((16,), i32)

        @pl.loop(0, rows_per)
        def _row(rl):
            r = wid * rows_per + rl
            r16 = jnp.broadcast_to(r, (16,))
            thr_s = plsc.load_gather(thr_v, [r16])

            @pl.loop(0, K_SEL // 16)
            def _z(z):
                sel_val[pl.ds(z * 16, 16)] = zf
                sel_idx[pl.ds(z * 16, 16)] = zi

            off_ref[...] = zi
            for w in range(nw):
                pltpu.sync_copy(
                    lat_hbm.at[r, pl.ds(w * _SC_ROWW, _SC_ROWW)], rowb)

                @pl.loop(0, _SC_ROWW // 16)
                def _scan(i):
                    v = rowb[pl.ds(i * 16, 16)]
                    msk = v >= thr_s
                    cs = plsc.cumsum(msk.astype(i32))
                    off = off_ref[...]
                    pos = off + cs - 1
                    ok = msk & (pos < K_SEL)
                    plsc.store_scatter(sel_val, [pos], v, mask=ok)
                    gidx = (w * _SC_ROWW) + i * 16 + ii
                    plsc.store_scatter(sel_idx, [pos], gidx, mask=ok)
                    off_ref[...] = off + plsc.all_reduce_population_count(msk)

            @pl.loop(0, nchunk)
            def _za(c):
                outv[pl.ds(c * 16, 16)] = zf

            for g in range(ng):
                pltpu.async_copy(
                    w_hbm.at[sel_idx.at[pl.ds(g * _SC_GW, _SC_GW)]],
                    wbuf, sem).wait()

                @pl.loop(0, _SC_GW)
                def _aj(j):
                    j16 = jnp.broadcast_to(g * _SC_GW + j, (16,))
                    vs = plsc.load_gather(sel_val, [j16])
                    jloc = jnp.broadcast_to(j, (16,))
                    for c in range(nchunk):
                        prod = vs * plsc.load_gather(wbuf, [jloc, c * 16 + ii])
                        plsc.addupdate(outv.at[pl.ds(c * 16, 16)], prod)

            mu_s = plsc.load_gather(mu_v, [r16])
            std_s = plsc.load_gather(std_v, [r16])

            @pl.loop(0, nchunk)
            def _fin(c):
                outv[pl.ds(c * 16, 16)] = (outv[pl.ds(c * 16, 16)] * std_s
                                           + mu_s)

            pltpu.sync_copy(outv, out_hbm.at[r])

    return sc_kernel(lat, thr, mu, std, W_enc)


def kernel(txt_x, W_enc):
    b, d = txt_x.shape
    h = W_enc.shape[0]
    th = 2048 if h % 2048 == 0 else h
    nt = h // th
    f32 = jnp.float32

    lat, mn, mx, mu, std = pl.pallas_call(
        _encode_body,
        grid=(nt,),
        in_specs=[
            pl.BlockSpec((b, d), lambda i: (0, 0)),
            pl.BlockSpec((th, d), lambda i: (i, 0)),
        ],
        out_specs=[
            pl.BlockSpec((b, th), lambda i: (0, i)),
            pl.BlockSpec((b, 1), lambda i: (0, 0)),
            pl.BlockSpec((b, 1), lambda i: (0, 0)),
            pl.BlockSpec((b, 1), lambda i: (0, 0)),
            pl.BlockSpec((b, 1), lambda i: (0, 0)),
        ],
        out_shape=[
            jax.ShapeDtypeStruct((b, h), f32),
            jax.ShapeDtypeStruct((b, 1), f32),
            jax.ShapeDtypeStruct((b, 1), f32),
            jax.ShapeDtypeStruct((b, 1), f32),
            jax.ShapeDtypeStruct((b, 1), f32),
        ],
        scratch_shapes=[pltpu.VMEM((b, d), f32)],
    )(txt_x, W_enc)

    thr = pl.pallas_call(
        _select_body,
        out_shape=jax.ShapeDtypeStruct((b, 1), f32),
    )(lat, mn, mx)

    recon = _sc_decode(lat, thr.reshape((b,)), mu.reshape((b,)),
                       std.reshape((b,)), W_enc)

    return (recon, lat)


# speedup vs baseline: 1.0523x; 1.0523x over previous
import dataclasses
import functools

import jax
import jax.numpy as jnp
from jax import lax
from jax.experimental import pallas as pl
from jax.experimental.pallas import tpu as pltpu
from jax.experimental.pallas import tpu_sc as plsc

K_SEL = 256
LN_EPS = 1e-5
_PREC = jax.lax.Precision.DEFAULT
_N_ITERS = 18


def _encode_body(x_ref, w_ref, lat_ref, mn_ref, mx_ref, mu_ref, std_ref,
                 xn_ref):
    step = pl.program_id(0)
    d = x_ref.shape[1]

    @pl.when(step == 0)
    def _():
        x = x_ref[...]
        mu = jnp.mean(x, axis=1, keepdims=True)
        xc = x - mu
        m2 = jnp.mean(xc, axis=1, keepdims=True)
        var = jnp.sum((xc - m2) * (xc - m2), axis=1, keepdims=True) / (d - 1)
        std = jnp.sqrt(var)
        xn_ref[...] = xc / (std + LN_EPS)
        mu_ref[...] = mu
        std_ref[...] = std

    w = w_ref[...]
    w2 = jnp.sum(w * w, axis=1, keepdims=True)
    inv = 1.0 / jnp.maximum(jnp.sqrt(w2), 1e-12)
    wn = w * inv
    lat = jax.lax.dot_general(
        xn_ref[...], wn, (((1,), (1,)), ((), ())),
        precision=_PREC, preferred_element_type=jnp.float32)
    lat_ref[...] = lat
    tmin = jnp.min(lat, axis=1, keepdims=True)
    tmax = jnp.max(lat, axis=1, keepdims=True)

    @pl.when(step == 0)
    def _():
        mn_ref[...] = tmin
        mx_ref[...] = tmax

    @pl.when(step > 0)
    def _():
        mn_ref[...] = jnp.minimum(mn_ref[...], tmin)
        mx_ref[...] = jnp.maximum(mx_ref[...], tmax)


def _select_body(lat_ref, mn_ref, mx_ref, thr_ref):
    b, h = lat_ref.shape
    n_chunks = 16
    ch = h // n_chunks

    def count_ge(t):
        c = jnp.zeros((b, 1), jnp.int32)
        for j in range(n_chunks):
            v = lat_ref[:, pl.ds(j * ch, ch)]
            c = c + jnp.sum((v >= t).astype(jnp.int32), axis=1,
                            keepdims=True)
        return c

    mn = mn_ref[...]
    mx = mx_ref[...]
    lo = mn
    hi = mx + (jnp.abs(mx) * 1e-6 + 1e-30)
    c_lo = jnp.full((b, 1), h, jnp.int32)
    c_hi = jnp.zeros((b, 1), jnp.int32)

    for _ in range(_N_ITERS):
        c_lo_f = c_lo.astype(jnp.float32)
        c_hi_f = jnp.maximum(c_hi.astype(jnp.float32), 0.7)
        frac = jnp.log(c_lo_f / K_SEL) / jnp.log(c_lo_f / c_hi_f)
        frac = jnp.clip(frac, 0.015625, 0.984375)
        m0 = lo + (hi - lo) * frac
        m = jnp.where((m0 > lo) & (m0 < hi), m0, 0.5 * (lo + hi))
        c = count_ge(m)
        go_hi = c <= K_SEL
        go_lo = c > K_SEL
        hi = jnp.where(go_hi, m, hi)
        c_hi = jnp.where(go_hi, c, c_hi)
        lo = jnp.where(go_lo, m, lo)
        c_lo = jnp.where(go_lo, c, c_lo)

    thr_ref[...] = hi


def _decode_body(lat_ref, w_ref, thr_ref, mu_ref, std_ref, out_ref, acc_ref):
    step = pl.program_id(0)
    nt = pl.num_programs(0)

    @pl.when(step == 0)
    def _():
        acc_ref[...] = jnp.zeros_like(acc_ref)

    lat = lat_ref[...]
    masked = jnp.where(lat >= thr_ref[...], lat, 0.0)
    acc_ref[...] += jax.lax.dot_general(
        masked, w_ref[...], (((1,), (0,)), ((), ())),
        precision=_PREC, preferred_element_type=jnp.float32)

    @pl.when(step == nt - 1)
    def _():
        out_ref[...] = acc_ref[...] * std_ref[...] + mu_ref[...]


_SC_ROWW = 16384
_SC_GW = 64


def _sc_accum(g, wbuf, sel_val, outv, ii, nchunk):
    bsel = jnp.full((16,), g % 2, jnp.int32)

    @pl.loop(0, _SC_GW)
    def _aj(j):
        j16 = jnp.broadcast_to(g * _SC_GW + j, (16,))
        vs = plsc.load_gather(sel_val, [j16])
        jloc = jnp.broadcast_to(j, (16,))
        for c in range(nchunk):
            prod = vs * plsc.load_gather(wbuf, [bsel, jloc, c * 16 + ii])
            plsc.addupdate(outv.at[pl.ds(c * 16, 16)], prod)


def _sc_decode(lat, thr, mu, std, W_enc):
    b, h = lat.shape
    d = W_enc.shape[1]
    f32 = jnp.float32
    i32 = jnp.int32
    mesh = plsc.VectorSubcoreMesh(core_axis_name="c", subcore_axis_name="s")
    n_tec = 32
    rows_per = b // n_tec
    nw = h // _SC_ROWW
    ng = K_SEL // _SC_GW
    nchunk = d // 16

    cp = pltpu.CompilerParams()
    if "needs_layout_passes" in pltpu.CompilerParams.__dataclass_fields__:
        cp = dataclasses.replace(cp, needs_layout_passes=False)

    @functools.partial(
        pl.kernel,
        out_type=jax.ShapeDtypeStruct((b, d), f32),
        mesh=mesh,
        compiler_params=cp,
        scratch_types=[
            pltpu.VMEM((_SC_ROWW,), f32),
            pltpu.VMEM((K_SEL,), f32),
            pltpu.VMEM((K_SEL,), i32),
            pltpu.VMEM((2, _SC_GW, d), f32),
            pltpu.VMEM((d,), f32),
            pltpu.VMEM((b,), f32),
            pltpu.VMEM((b,), f32),
            pltpu.VMEM((b,), f32),
            pltpu.SemaphoreType.DMA,
        ],
    )
    def sc_kernel(lat_hbm, thr_hbm, mu_hbm, std_hbm, w_hbm, out_hbm,
                  rowb, sel_val, sel_idx, wbuf, outv, thr_v, mu_v, std_v,
                  sem):
        wid = lax.axis_index("s") * 2 + lax.axis_index("c")
        pltpu.sync_copy(thr_hbm, thr_v)
        pltpu.sync_copy(mu_hbm, mu_v)
        pltpu.sync_copy(std_hbm, std_v)
        ii = lax.iota(i32, 16)
        zf = jnp.zeros((16,), f32)
        zi = jnp.zeros((16,), i32)

        @pl.loop(0, rows_per)
        def _row(rl):
            r = wid * rows_per + rl
            r16 = jnp.broadcast_to(r, (16,))
            thr_s = plsc.load_gather(thr_v, [r16])

            @pl.loop(0, K_SEL // 16)
            def _z(z):
                sel_val[pl.ds(z * 16, 16)] = zf
                sel_idx[pl.ds(z * 16, 16)] = zi

            off = zi
            for w in range(nw):
                pltpu.sync_copy(
                    lat_hbm.at[r, pl.ds(w * _SC_ROWW, _SC_ROWW)], rowb)

                def _scan(i, off):
                    v = rowb[pl.ds(i * 16, 16)]
                    msk = v >= thr_s
                    cs = plsc.cumsum(msk.astype(i32))
                    pos = off + cs - 1
                    ok = msk & (pos < K_SEL)
                    plsc.store_scatter(sel_val, [pos], v, mask=ok)
                    gidx = (w * _SC_ROWW) + i * 16 + ii
                    plsc.store_scatter(sel_idx, [pos], gidx, mask=ok)
                    return off + plsc.all_reduce_population_count(msk)

                off = lax.fori_loop(0, _SC_ROWW // 16, _scan, off,
                                    unroll=4)

            @pl.loop(0, nchunk)
            def _za(c):
                outv[pl.ds(c * 16, 16)] = zf

            copies = []
            for g in range(ng):
                copies.append(pltpu.async_copy(
                    w_hbm.at[sel_idx.at[pl.ds(g * _SC_GW, _SC_GW)]],
                    wbuf.at[g % 2], sem))
                if g > 0:
                    copies[g - 1].wait()
                    _sc_accum(g - 1, wbuf, sel_val, outv, ii, nchunk)
            copies[ng - 1].wait()
            _sc_accum(ng - 1, wbuf, sel_val, outv, ii, nchunk)

            mu_s = plsc.load_gather(mu_v, [r16])
            std_s = plsc.load_gather(std_v, [r16])

            @pl.loop(0, nchunk)
            def _fin(c):
                outv[pl.ds(c * 16, 16)] = (outv[pl.ds(c * 16, 16)] * std_s
                                           + mu_s)

            pltpu.sync_copy(outv, out_hbm.at[r])

    return sc_kernel(lat, thr, mu, std, W_enc)


def kernel(txt_x, W_enc):
    b, d = txt_x.shape
    h = W_enc.shape[0]
    th = 2048 if h % 2048 == 0 else h
    nt = h // th
    f32 = jnp.float32

    lat, mn, mx, mu, std = pl.pallas_call(
        _encode_body,
        grid=(nt,),
        in_specs=[
            pl.BlockSpec((b, d), lambda i: (0, 0)),
            pl.BlockSpec((th, d), lambda i: (i, 0)),
        ],
        out_specs=[
            pl.BlockSpec((b, th), lambda i: (0, i)),
            pl.BlockSpec((b, 1), lambda i: (0, 0)),
            pl.BlockSpec((b, 1), lambda i: (0, 0)),
            pl.BlockSpec((b, 1), lambda i: (0, 0)),
            pl.BlockSpec((b, 1), lambda i: (0, 0)),
        ],
        out_shape=[
            jax.ShapeDtypeStruct((b, h), f32),
            jax.ShapeDtypeStruct((b, 1), f32),
            jax.ShapeDtypeStruct((b, 1), f32),
            jax.ShapeDtypeStruct((b, 1), f32),
            jax.ShapeDtypeStruct((b, 1), f32),
        ],
        scratch_shapes=[pltpu.VMEM((b, d), f32)],
    )(txt_x, W_enc)

    thr = pl.pallas_call(
        _select_body,
        out_shape=jax.ShapeDtypeStruct((b, 1), f32),
    )(lat, mn, mx)

    recon = _sc_decode(lat, thr.reshape((b,)), mu.reshape((b,)),
                       std.reshape((b,)), W_enc)

    return (recon, lat)
